# baseline (speedup 1.0000x reference)
import functools

import jax
import jax.numpy as jnp
from jax import lax
from jax.experimental import pallas as pl
from jax.experimental.pallas import tpu as pltpu
from jax.experimental.pallas import tpu_sc as plsc

_B, _L, _E, _V = 16384, 200, 8, 10
_TOT = _B * _L
_RW = 128
_NROW = _TOT // _RW
_NC, _NS = 2, 16
_NW = _NC * _NS
_RPW = _NROW // _NW
_C = 25
_NCH = _RPW // _C

_mesh = plsc.VectorSubcoreMesh(core_axis_name="c", subcore_axis_name="s")


@functools.partial(
    pl.kernel,
    out_type=[
        jax.ShapeDtypeStruct((_NROW * _E, _RW), jnp.float32),
        jax.ShapeDtypeStruct((_NW, 16), jnp.float32),
    ],
    mesh=_mesh,
    compiler_params=pltpu.CompilerParams(
        needs_layout_passes=False, use_tc_tiling_on_sc=False),
    scratch_types=[
        pltpu.VMEM((_C, _RW), jnp.int32),
        pltpu.VMEM((_C, _RW), jnp.int32),
        pltpu.VMEM((_C * _E, _RW), jnp.float32),
        pltpu.VMEM((_C * _E, _RW), jnp.float32),
        pltpu.VMEM((128,), jnp.float32),
        pltpu.VMEM((16,), jnp.float32),
        pltpu.VMEM((16,), jnp.float32),
        pltpu.SemaphoreType.DMA,
        pltpu.SemaphoreType.DMA,
        pltpu.SemaphoreType.DMA,
        pltpu.SemaphoreType.DMA,
    ],
)
def _sc_gather(ids_hbm, wflat_hbm, out_hbm, part_hbm,
               idx0, idx1, rows0, rows1, wflat_v, rs_v, acc_v,
               sem_i0, sem_i1, sem_o0, sem_o1):
    cid = lax.axis_index("c")
    sid = lax.axis_index("s")
    wid = sid * _NC + cid
    base = wid * _RPW

    idx = (idx0, idx1)
    rows = (rows0, rows1)
    sem_i = (sem_i0, sem_i1)
    sem_o = (sem_o0, sem_o1)

    pltpu.sync_copy(wflat_hbm, wflat_v)
    lanes = lax.iota(jnp.int32, 16)
    rs = jnp.zeros((16,), jnp.float32)
    for d in range(_E):
        rs = rs + plsc.load_gather(wflat_v, [lanes * _E + d])
    rs_v[...] = rs

    def ids_start(i, b):
        pltpu.async_copy(ids_hbm.at[pl.ds(base + i * _C, _C)], idx[b], sem_i[b])

    def ids_wait(b):
        pltpu.make_async_copy(
            ids_hbm.at[pl.ds(0, _C)], idx[b], sem_i[b]).wait()

    def out_start(i, b):
        pltpu.async_copy(rows[b],
                         out_hbm.at[pl.ds((base + i * _C) * _E, _C * _E)],
                         sem_o[b])

    def out_wait(b):
        pltpu.make_async_copy(
            rows[b], out_hbm.at[pl.ds(0, _C * _E)], sem_o[b]).wait()

    def chunk_mid(b, acc):
        ids_wait(b)

        def jbody(j, a):
            for k in range(_RW // 16):
                iv = idx[b][j, pl.ds(k * 16, 16)]
                a = a + plsc.load_gather(rs_v, [iv])
                iv8 = iv * _E
                for e in range(_E):
                    v = plsc.load_gather(wflat_v, [iv8 + e])
                    rows[b][j * _E + e, pl.ds(k * 16, 16)] = v
            return a

        return plsc.parallel_loop(0, _C, 1, unroll=5, carry=acc)(jbody)

    ids_start(0, 0)
    ids_start(1, 1)
    acc = jnp.zeros((16,), jnp.float32)
    for b in (0, 1):
        acc = chunk_mid(b, acc)
        ids_start(b + 2, b)
        out_start(b, b)

    def macro(m, acc):
        for b in (0, 1):
            i = 2 * m + b
            out_wait(b)
            acc = chunk_mid(b, acc)
            nxt = jnp.where(i + 2 < _NCH, i + 2, 0)
            ids_start(nxt, b)
            out_start(i, b)
        return acc

    acc = lax.fori_loop(1, _NCH // 2, macro, acc)

    for b in (0, 1):
        ids_wait(b)
        out_wait(b)

    acc_v[...] = acc
    pltpu.sync_copy(acc_v, part_hbm.at[wid])


def _loss_body(p_ref, o_ref):
    o_ref[0, 0] = jnp.sum(p_ref[...]) * (1.0 / (_TOT * _E))


_tc_loss = pl.pallas_call(
    _loss_body,
    out_shape=jax.ShapeDtypeStruct((1, 1), jnp.float32),
    out_specs=pl.BlockSpec(memory_space=pltpu.SMEM),
)


def kernel(input_ids, W):
    ids = input_ids.astype(jnp.int32).T.reshape(_NROW, _RW)
    w = W.astype(jnp.float32)
    wflat = jnp.pad(w.reshape(-1), (0, 128 - _V * _E))
    out2, part = _sc_gather(ids, wflat)
    o4 = out2.reshape(_L, _B // _RW, _E, _RW)
    outputs = o4.transpose(1, 3, 0, 2).reshape(_B, _L, _E)
    loss = _tc_loss(part)[0, 0]
    return (loss, outputs)


# speedup vs baseline: 1.3280x; 1.3280x over previous
import functools

import jax
import jax.numpy as jnp
from jax import lax
from jax.experimental import pallas as pl
from jax.experimental.pallas import tpu as pltpu
from jax.experimental.pallas import tpu_sc as plsc

_B, _L, _E, _V = 16384, 200, 8, 10
_TOT = _B * _L
_RW = 128
_NROW = _TOT // _RW
_NC, _NS = 2, 16
_NW = _NC * _NS
_RPW = _NROW // _NW
_C = 16
_NCH = _RPW // _C

_mesh = plsc.VectorSubcoreMesh(core_axis_name="c", subcore_axis_name="s")


@functools.partial(
    pl.kernel,
    out_type=[
        jax.ShapeDtypeStruct((_NROW * _E, _RW), jnp.float32),
        jax.ShapeDtypeStruct((_NW, 16), jnp.float32),
    ],
    mesh=_mesh,
    compiler_params=pltpu.CompilerParams(
        needs_layout_passes=False, use_tc_tiling_on_sc=False),
    scratch_types=[
        pltpu.VMEM((_C, _RW), jnp.int32),
        pltpu.VMEM((_C, _RW), jnp.int32),
        pltpu.VMEM((_C * _E, _RW), jnp.float32),
        pltpu.VMEM((_C * _E, _RW), jnp.float32),
        pltpu.VMEM((128,), jnp.float32),
        pltpu.VMEM((16,), jnp.float32),
        pltpu.VMEM((16,), jnp.float32),
        pltpu.SemaphoreType.DMA,
        pltpu.SemaphoreType.DMA,
        pltpu.SemaphoreType.DMA,
        pltpu.SemaphoreType.DMA,
    ],
)
def _sc_gather(ids_hbm, wflat_hbm, out_hbm, part_hbm,
               idx0, idx1, rows0, rows1, wflat_v, rs_v, acc_v,
               sem_i0, sem_i1, sem_o0, sem_o1):
    cid = lax.axis_index("c")
    sid = lax.axis_index("s")
    wid = sid * _NC + cid
    base = wid * _RPW

    idx = (idx0, idx1)
    rows = (rows0, rows1)
    sem_i = (sem_i0, sem_i1)
    sem_o = (sem_o0, sem_o1)

    pltpu.sync_copy(wflat_hbm, wflat_v)
    lanes = lax.iota(jnp.int32, 16)
    rs = jnp.zeros((16,), jnp.float32)
    for d in range(_E):
        rs = rs + plsc.load_gather(wflat_v, [lanes * _E + d])
    rs_v[...] = rs

    def ids_start(i, b):
        pltpu.async_copy(ids_hbm.at[pl.ds(base + i * _C, _C)], idx[b], sem_i[b])

    def ids_wait(b):
        pltpu.make_async_copy(
            ids_hbm.at[pl.ds(0, _C)], idx[b], sem_i[b]).wait()

    def out_start(i, b):
        pltpu.async_copy(rows[b],
                         out_hbm.at[pl.ds((base + i * _C) * _E, _C * _E)],
                         sem_o[b])

    def out_wait(b):
        pltpu.make_async_copy(
            rows[b], out_hbm.at[pl.ds(0, _C * _E)], sem_o[b]).wait()

    def chunk_mid(b, acc):
        ids_wait(b)

        def jbody(j, a):
            for k in range(_RW // 16):
                iv = idx[b][j, pl.ds(k * 16, 16)]
                a = a + plsc.load_gather(rs_v, [iv])
                iv8 = iv * _E
                for e in range(_E):
                    v = plsc.load_gather(wflat_v, [iv8 + e])
                    rows[b][j * _E + e, pl.ds(k * 16, 16)] = v
            return a

        return plsc.parallel_loop(0, _C, 1, unroll=2, carry=acc)(jbody)

    ids_start(0, 0)
    ids_start(1, 1)
    acc = jnp.zeros((16,), jnp.float32)
    for b in (0, 1):
        acc = chunk_mid(b, acc)
        ids_start(b + 2, b)
        out_start(b, b)

    def macro(m, acc):
        for b in (0, 1):
            i = 2 * m + b
            out_wait(b)
            acc = chunk_mid(b, acc)
            nxt = jnp.where(i + 2 < _NCH, i + 2, 0)
            ids_start(nxt, b)
            out_start(i, b)
        return acc

    acc = lax.fori_loop(1, _NCH // 2, macro, acc)

    for b in (0, 1):
        ids_wait(b)
        out_wait(b)

    acc_v[...] = acc
    pltpu.sync_copy(acc_v, part_hbm.at[wid])


def _loss_body(p_ref, o_ref):
    o_ref[0, 0] = jnp.sum(p_ref[...]) * (1.0 / (_TOT * _E))


_tc_loss = pl.pallas_call(
    _loss_body,
    out_shape=jax.ShapeDtypeStruct((1, 1), jnp.float32),
    out_specs=pl.BlockSpec(memory_space=pltpu.SMEM),
)


def kernel(input_ids, W):
    ids = input_ids.astype(jnp.int32).T.reshape(_NROW, _RW)
    w = W.astype(jnp.float32)
    wflat = jnp.pad(w.reshape(-1), (0, 128 - _V * _E))
    out2, part = _sc_gather(ids, wflat)
    o4 = out2.reshape(_L, _B // _RW, _E, _RW)
    outputs = o4.transpose(1, 3, 0, 2).reshape(_B, _L, _E)
    loss = _tc_loss(part)[0, 0]
    return (loss, outputs)


# speedup vs baseline: 1.6721x; 1.2592x over previous
import functools

import jax
import jax.numpy as jnp
from jax import lax
from jax.experimental import pallas as pl
from jax.experimental.pallas import tpu as pltpu
from jax.experimental.pallas import tpu_sc as plsc

_B, _L, _E, _V = 16384, 200, 8, 10
_TOT = _B * _L
_RW = 128
_NROW = _TOT // _RW
_NC, _NS = 2, 16
_NW = _NC * _NS
_RPW = _NROW // _NW
_C = 16
_NCH = _RPW // _C

_mesh = plsc.VectorSubcoreMesh(core_axis_name="c", subcore_axis_name="s")


@functools.partial(
    pl.kernel,
    out_type=[
        jax.ShapeDtypeStruct((_NROW * _E, _RW), jnp.float32),
        jax.ShapeDtypeStruct((_NW, 16), jnp.float32),
    ],
    mesh=_mesh,
    compiler_params=pltpu.CompilerParams(
        needs_layout_passes=False, use_tc_tiling_on_sc=False),
    scratch_types=[
        pltpu.VMEM((_C, _RW), jnp.int32),
        pltpu.VMEM((_C, _RW), jnp.int32),
        pltpu.VMEM((_C * _E, _RW), jnp.float32),
        pltpu.VMEM((_C * _E, _RW), jnp.float32),
        pltpu.VMEM((128,), jnp.float32),
        pltpu.VMEM((_V * _E * 16,), jnp.float32),
        pltpu.VMEM((_V * 16,), jnp.float32),
        pltpu.VMEM((16,), jnp.float32),
        pltpu.VMEM((16,), jnp.float32),
        pltpu.SemaphoreType.DMA,
        pltpu.SemaphoreType.DMA,
        pltpu.SemaphoreType.DMA,
        pltpu.SemaphoreType.DMA,
    ],
)
def _sc_gather(ids_hbm, wflat_hbm, out_hbm, part_hbm,
               idx0, idx1, rows0, rows1, wflat_v, wbank, rsbank, rs_v, acc_v,
               sem_i0, sem_i1, sem_o0, sem_o1):
    cid = lax.axis_index("c")
    sid = lax.axis_index("s")
    wid = sid * _NC + cid
    base = wid * _RPW

    idx = (idx0, idx1)
    rows = (rows0, rows1)
    sem_i = (sem_i0, sem_i1)
    sem_o = (sem_o0, sem_o1)

    pltpu.sync_copy(wflat_hbm, wflat_v)
    lanes = lax.iota(jnp.int32, 16)
    rs = jnp.zeros((16,), jnp.float32)
    for d in range(_E):
        rs = rs + plsc.load_gather(wflat_v, [lanes * _E + d])
    rs_v[...] = rs
    for g in range(_V * _E):
        wbank[pl.ds(g * 16, 16)] = plsc.load_gather(
            wflat_v, [jnp.full((16,), g, jnp.int32)])
    for t in range(_V):
        rsbank[pl.ds(t * 16, 16)] = plsc.load_gather(
            rs_v, [jnp.full((16,), t, jnp.int32)])

    def ids_start(i, b):
        pltpu.async_copy(ids_hbm.at[pl.ds(base + i * _C, _C)], idx[b], sem_i[b])

    def ids_wait(b):
        pltpu.make_async_copy(
            ids_hbm.at[pl.ds(0, _C)], idx[b], sem_i[b]).wait()

    def out_start(i, b):
        pltpu.async_copy(rows[b],
                         out_hbm.at[pl.ds((base + i * _C) * _E, _C * _E)],
                         sem_o[b])

    def out_wait(b):
        pltpu.make_async_copy(
            rows[b], out_hbm.at[pl.ds(0, _C * _E)], sem_o[b]).wait()

    def chunk_mid(b, acc):
        ids_wait(b)

        def jbody(j, a):
            for k in range(_RW // 16):
                iv = idx[b][j, pl.ds(k * 16, 16)]
                a = a + plsc.load_gather(rsbank, [iv * 16 + lanes])
                iv128 = iv * (_E * 16)
                for e in range(_E):
                    v = plsc.load_gather(wbank, [iv128 + (lanes + e * 16)])
                    rows[b][j * _E + e, pl.ds(k * 16, 16)] = v
            return a

        return plsc.parallel_loop(0, _C, 1, carry=acc)(jbody)

    ids_start(0, 0)
    ids_start(1, 1)
    acc = jnp.zeros((16,), jnp.float32)
    for b in (0, 1):
        acc = chunk_mid(b, acc)
        ids_start(b + 2, b)
        out_start(b, b)

    def macro(m, acc):
        for b in (0, 1):
            i = 2 * m + b
            out_wait(b)
            acc = chunk_mid(b, acc)
            nxt = jnp.where(i + 2 < _NCH, i + 2, 0)
            ids_start(nxt, b)
            out_start(i, b)
        return acc

    acc = lax.fori_loop(1, _NCH // 2, macro, acc)

    for b in (0, 1):
        ids_wait(b)
        out_wait(b)

    acc_v[...] = acc
    pltpu.sync_copy(acc_v, part_hbm.at[wid])


def _loss_body(p_ref, o_ref):
    o_ref[0, 0] = jnp.sum(p_ref[...]) * (1.0 / (_TOT * _E))


_tc_loss = pl.pallas_call(
    _loss_body,
    out_shape=jax.ShapeDtypeStruct((1, 1), jnp.float32),
    out_specs=pl.BlockSpec(memory_space=pltpu.SMEM),
)


def kernel(input_ids, W):
    ids = input_ids.astype(jnp.int32).T.reshape(_NROW, _RW)
    w = W.astype(jnp.float32)
    wflat = jnp.pad(w.reshape(-1), (0, 128 - _V * _E))
    out2, part = _sc_gather(ids, wflat)
    o4 = out2.reshape(_L, _B // _RW, _E, _RW)
    outputs = o4.transpose(1, 3, 0, 2).reshape(_B, _L, _E)
    loss = _tc_loss(part)[0, 0]
    return (loss, outputs)


# speedup vs baseline: 1.7085x; 1.0218x over previous
import functools

import jax
import jax.numpy as jnp
from jax import lax
from jax.experimental import pallas as pl
from jax.experimental.pallas import tpu as pltpu
from jax.experimental.pallas import tpu_sc as plsc

_B, _L, _E, _V = 16384, 200, 8, 10
_TOT = _B * _L
_RW = 128
_NROW = _TOT // _RW
_NC, _NS = 2, 16
_NW = _NC * _NS
_RPW = _NROW // _NW
_C = 16
_NCH = _RPW // _C

_mesh = plsc.VectorSubcoreMesh(core_axis_name="c", subcore_axis_name="s")


@functools.partial(
    pl.kernel,
    out_type=[
        jax.ShapeDtypeStruct((_NROW * _E, _RW), jnp.float32),
        jax.ShapeDtypeStruct((_NW, 16), jnp.float32),
    ],
    mesh=_mesh,
    compiler_params=pltpu.CompilerParams(
        needs_layout_passes=False, use_tc_tiling_on_sc=False),
    scratch_types=[
        pltpu.VMEM((_C, _RW), jnp.int32),
        pltpu.VMEM((_C, _RW), jnp.int32),
        pltpu.VMEM((_C * _E, _RW), jnp.float32),
        pltpu.VMEM((_C * _E, _RW), jnp.float32),
        pltpu.VMEM((_V * _E * 16,), jnp.float32),
        pltpu.VMEM((_V * 16,), jnp.float32),
        pltpu.VMEM((16,), jnp.float32),
        pltpu.SemaphoreType.DMA,
        pltpu.SemaphoreType.DMA,
        pltpu.SemaphoreType.DMA,
        pltpu.SemaphoreType.DMA,
    ],
)
def _sc_gather(ids_hbm, wbank_hbm, rsbank_hbm, out_hbm, part_hbm,
               idx0, idx1, rows0, rows1, wbank, rsbank, acc_v,
               sem_i0, sem_i1, sem_o0, sem_o1):
    cid = lax.axis_index("c")
    sid = lax.axis_index("s")
    wid = sid * _NC + cid
    base = wid * _RPW

    idx = (idx0, idx1)
    rows = (rows0, rows1)
    sem_i = (sem_i0, sem_i1)
    sem_o = (sem_o0, sem_o1)

    lanes = lax.iota(jnp.int32, 16)
    pltpu.sync_copy(wbank_hbm, wbank)
    pltpu.sync_copy(rsbank_hbm, rsbank)

    def ids_start(i, b):
        pltpu.async_copy(ids_hbm.at[pl.ds(base + i * _C, _C)], idx[b], sem_i[b])

    def ids_wait(b):
        pltpu.make_async_copy(
            ids_hbm.at[pl.ds(0, _C)], idx[b], sem_i[b]).wait()

    def out_start(i, b):
        pltpu.async_copy(rows[b],
                         out_hbm.at[pl.ds((base + i * _C) * _E, _C * _E)],
                         sem_o[b])

    def out_wait(b):
        pltpu.make_async_copy(
            rows[b], out_hbm.at[pl.ds(0, _C * _E)], sem_o[b]).wait()

    def chunk_mid(b, acc):
        ids_wait(b)

        def jbody(j, a):
            for k in range(_RW // 16):
                iv = idx[b][j, pl.ds(k * 16, 16)]
                a = a + plsc.load_gather(rsbank, [iv * 16 + lanes])
                iv128 = iv * (_E * 16)
                for e in range(_E):
                    v = plsc.load_gather(wbank, [iv128 + (lanes + e * 16)])
                    rows[b][j * _E + e, pl.ds(k * 16, 16)] = v
            return a

        return plsc.parallel_loop(0, _C, 1, carry=acc)(jbody)

    ids_start(0, 0)
    ids_start(1, 1)
    acc = jnp.zeros((16,), jnp.float32)
    for b in (0, 1):
        acc = chunk_mid(b, acc)
        ids_start(b + 2, b)
        out_start(b, b)

    def macro(m, acc):
        for b in (0, 1):
            i = 2 * m + b
            out_wait(b)
            acc = chunk_mid(b, acc)
            nxt = jnp.where(i + 2 < _NCH, i + 2, 0)
            ids_start(nxt, b)
            out_start(i, b)
        return acc

    acc = lax.fori_loop(1, _NCH // 2, macro, acc)

    for b in (0, 1):
        ids_wait(b)
        out_wait(b)

    acc_v[...] = acc
    pltpu.sync_copy(acc_v, part_hbm.at[wid])


def _loss_body(p_ref, o_ref):
    o_ref[0, 0] = jnp.sum(p_ref[...]) * (1.0 / (_TOT * _E))


_tc_loss = pl.pallas_call(
    _loss_body,
    out_shape=jax.ShapeDtypeStruct((1, 1), jnp.float32),
    out_specs=pl.BlockSpec(memory_space=pltpu.SMEM),
)


def kernel(input_ids, W):
    ids = input_ids.astype(jnp.int32).T.reshape(_NROW, _RW)
    w = W.astype(jnp.float32)
    wbank = jnp.repeat(w.reshape(-1), 16)
    rsbank = jnp.repeat(w.sum(axis=1), 16)
    out2, part = _sc_gather(ids, wbank, rsbank)
    o4 = out2.reshape(_L, _B // _RW, _E, _RW)
    outputs = o4.transpose(1, 3, 0, 2).reshape(_B, _L, _E)
    loss = _tc_loss(part)[0, 0]
    return (loss, outputs)


# speedup vs baseline: 1.8336x; 1.0732x over previous
import functools

import jax
import jax.numpy as jnp
from jax import lax
from jax.experimental import pallas as pl
from jax.experimental.pallas import tpu as pltpu
from jax.experimental.pallas import tpu_sc as plsc

_B, _L, _E, _V = 16384, 200, 8, 10
_TOT = _B * _L
_RW = 128
_NROW = _TOT // _RW
_NC, _NS = 2, 16
_NW = _NC * _NS
_RPW = _NROW // _NW
_C = 20
_NCH = _RPW // _C

_mesh = plsc.VectorSubcoreMesh(core_axis_name="c", subcore_axis_name="s")


@functools.partial(
    pl.kernel,
    out_type=[
        jax.ShapeDtypeStruct((_NROW * _E, _RW), jnp.float32),
        jax.ShapeDtypeStruct((_NW, 16), jnp.float32),
    ],
    mesh=_mesh,
    compiler_params=pltpu.CompilerParams(
        needs_layout_passes=False, use_tc_tiling_on_sc=False),
    scratch_types=[
        pltpu.VMEM((_C, _RW), jnp.int32),
        pltpu.VMEM((_C, _RW), jnp.int32),
        pltpu.VMEM((_C * _E, _RW), jnp.float32),
        pltpu.VMEM((_C * _E, _RW), jnp.float32),
        pltpu.VMEM((_V * _E * 16,), jnp.float32),
        pltpu.VMEM((_V * 16,), jnp.float32),
        pltpu.VMEM((16,), jnp.float32),
        pltpu.SemaphoreType.DMA,
        pltpu.SemaphoreType.DMA,
        pltpu.SemaphoreType.DMA,
        pltpu.SemaphoreType.DMA,
    ],
)
def _sc_gather(ids_hbm, wbank_hbm, rsbank_hbm, out_hbm, part_hbm,
               idx0, idx1, rows0, rows1, wbank, rsbank, acc_v,
               sem_i0, sem_i1, sem_o0, sem_o1):
    cid = lax.axis_index("c")
    sid = lax.axis_index("s")
    wid = sid * _NC + cid
    base = wid * _RPW

    idx = (idx0, idx1)
    rows = (rows0, rows1)
    sem_i = (sem_i0, sem_i1)
    sem_o = (sem_o0, sem_o1)

    lanes = lax.iota(jnp.int32, 16)
    pltpu.sync_copy(wbank_hbm, wbank)
    pltpu.sync_copy(rsbank_hbm, rsbank)

    def ids_start(i, b):
        pltpu.async_copy(ids_hbm.at[pl.ds(base + i * _C, _C)], idx[b], sem_i[b])

    def ids_wait(b):
        pltpu.make_async_copy(
            ids_hbm.at[pl.ds(0, _C)], idx[b], sem_i[b]).wait()

    def out_start(i, b):
        pltpu.async_copy(rows[b],
                         out_hbm.at[pl.ds((base + i * _C) * _E, _C * _E)],
                         sem_o[b])

    def out_wait(b):
        pltpu.make_async_copy(
            rows[b], out_hbm.at[pl.ds(0, _C * _E)], sem_o[b]).wait()

    def chunk_mid(b, acc):
        ids_wait(b)

        def jbody(j, a):
            for k in range(_RW // 16):
                iv = idx[b][j, pl.ds(k * 16, 16)]
                a = a + plsc.load_gather(rsbank, [iv * 16 + lanes])
                iv128 = iv * (_E * 16)
                for e in range(_E):
                    v = plsc.load_gather(wbank, [iv128 + (lanes + e * 16)])
                    rows[b][j * _E + e, pl.ds(k * 16, 16)] = v
            return a

        return plsc.parallel_loop(0, _C, 1, carry=acc)(jbody)

    ids_start(0, 0)
    ids_start(1, 1)
    acc = jnp.zeros((16,), jnp.float32)
    for b in (0, 1):
        acc = chunk_mid(b, acc)
        ids_start(b + 2, b)
        out_start(b, b)

    def macro(m, acc):
        for b in (0, 1):
            i = 2 * m + b
            out_wait(b)
            acc = chunk_mid(b, acc)
            nxt = jnp.where(i + 2 < _NCH, i + 2, 0)
            ids_start(nxt, b)
            out_start(i, b)
        return acc

    acc = lax.fori_loop(1, _NCH // 2, macro, acc)

    for b in (0, 1):
        ids_wait(b)
        out_wait(b)

    acc_v[...] = acc
    pltpu.sync_copy(acc_v, part_hbm.at[wid])


def _loss_body(p_ref, o_ref):
    o_ref[0, 0] = jnp.sum(p_ref[...]) * (1.0 / (_TOT * _E))


_tc_loss = pl.pallas_call(
    _loss_body,
    out_shape=jax.ShapeDtypeStruct((1, 1), jnp.float32),
    out_specs=pl.BlockSpec(memory_space=pltpu.SMEM),
)


def kernel(input_ids, W):
    ids = input_ids.astype(jnp.int32).T.reshape(_NROW, _RW)
    w = W.astype(jnp.float32)
    wbank = jnp.repeat(w.reshape(-1), 16)
    rsbank = jnp.repeat(w.sum(axis=1), 16)
    out2, part = _sc_gather(ids, wbank, rsbank)
    o4 = out2.reshape(_L, _B // _RW, _E, _RW)
    outputs = o4.transpose(1, 3, 0, 2).reshape(_B, _L, _E)
    loss = _tc_loss(part)[0, 0]
    return (loss, outputs)


# speedup vs baseline: 1.9527x; 1.0650x over previous
import functools

import jax
import jax.numpy as jnp
from jax import lax
from jax.experimental import pallas as pl
from jax.experimental.pallas import tpu as pltpu
from jax.experimental.pallas import tpu_sc as plsc

_B, _L, _E, _V = 16384, 200, 8, 10
_TOT = _B * _L
_RW = 128
_NROW = _TOT // _RW
_NC, _NS = 2, 16
_NW = _NC * _NS
_RPW = _NROW // _NW
_C = 25
_NCH = _RPW // _C

_mesh = plsc.VectorSubcoreMesh(core_axis_name="c", subcore_axis_name="s")


@functools.partial(
    pl.kernel,
    out_type=[
        jax.ShapeDtypeStruct((_NROW * _E, _RW), jnp.float32),
        jax.ShapeDtypeStruct((_NW, 16), jnp.float32),
    ],
    mesh=_mesh,
    compiler_params=pltpu.CompilerParams(
        needs_layout_passes=False, use_tc_tiling_on_sc=False),
    scratch_types=[
        pltpu.VMEM((_C, _RW), jnp.int32),
        pltpu.VMEM((_C, _RW), jnp.int32),
        pltpu.VMEM((_C * _E, _RW), jnp.float32),
        pltpu.VMEM((_C * _E, _RW), jnp.float32),
        pltpu.VMEM((_V * _E * 16,), jnp.float32),
        pltpu.VMEM((_V * 16,), jnp.float32),
        pltpu.VMEM((16,), jnp.float32),
        pltpu.SemaphoreType.DMA,
        pltpu.SemaphoreType.DMA,
        pltpu.SemaphoreType.DMA,
        pltpu.SemaphoreType.DMA,
    ],
)
def _sc_gather(ids_hbm, wbank_hbm, rsbank_hbm, out_hbm, part_hbm,
               idx0, idx1, rows0, rows1, wbank, rsbank, acc_v,
               sem_i0, sem_i1, sem_o0, sem_o1):
    cid = lax.axis_index("c")
    sid = lax.axis_index("s")
    wid = sid * _NC + cid
    base = wid * _RPW

    idx = (idx0, idx1)
    rows = (rows0, rows1)
    sem_i = (sem_i0, sem_i1)
    sem_o = (sem_o0, sem_o1)

    lanes = lax.iota(jnp.int32, 16)
    pltpu.sync_copy(wbank_hbm, wbank)
    pltpu.sync_copy(rsbank_hbm, rsbank)

    def ids_start(i, b):
        pltpu.async_copy(ids_hbm.at[pl.ds(base + i * _C, _C)], idx[b], sem_i[b])

    def ids_wait(b):
        pltpu.make_async_copy(
            ids_hbm.at[pl.ds(0, _C)], idx[b], sem_i[b]).wait()

    def out_start(i, b):
        pltpu.async_copy(rows[b],
                         out_hbm.at[pl.ds((base + i * _C) * _E, _C * _E)],
                         sem_o[b])

    def out_wait(b):
        pltpu.make_async_copy(
            rows[b], out_hbm.at[pl.ds(0, _C * _E)], sem_o[b]).wait()

    def chunk_mid(b, acc):
        ids_wait(b)

        def jbody(j, a):
            for k in range(_RW // 16):
                iv = idx[b][j, pl.ds(k * 16, 16)]
                a = a + plsc.load_gather(rsbank, [iv * 16 + lanes])
                iv128 = iv * (_E * 16)
                for e in range(_E):
                    v = plsc.load_gather(wbank, [iv128 + (lanes + e * 16)])
                    rows[b][j * _E + e, pl.ds(k * 16, 16)] = v
            return a

        return plsc.parallel_loop(0, _C, 1, carry=acc)(jbody)

    ids_start(0, 0)
    ids_start(1, 1)
    acc = jnp.zeros((16,), jnp.float32)
    for b in (0, 1):
        acc = chunk_mid(b, acc)
        ids_start(b + 2, b)
        out_start(b, b)

    def macro(m, acc):
        for b in (0, 1):
            i = 2 * m + b
            out_wait(b)
            acc = chunk_mid(b, acc)
            nxt = jnp.where(i + 2 < _NCH, i + 2, 0)
            ids_start(nxt, b)
            out_start(i, b)
        return acc

    acc = lax.fori_loop(1, _NCH // 2, macro, acc)

    for b in (0, 1):
        ids_wait(b)
        out_wait(b)

    acc_v[...] = acc
    pltpu.sync_copy(acc_v, part_hbm.at[wid])


def _loss_body(p_ref, o_ref):
    o_ref[0, 0] = jnp.sum(p_ref[...]) * (1.0 / (_TOT * _E))


_tc_loss = pl.pallas_call(
    _loss_body,
    out_shape=jax.ShapeDtypeStruct((1, 1), jnp.float32),
    out_specs=pl.BlockSpec(memory_space=pltpu.SMEM),
)


def kernel(input_ids, W):
    ids = input_ids.astype(jnp.int32).T.reshape(_NROW, _RW)
    w = W.astype(jnp.float32)
    wbank = jnp.repeat(w.reshape(-1), 16)
    rsbank = jnp.repeat(w.sum(axis=1), 16)
    out2, part = _sc_gather(ids, wbank, rsbank)
    o4 = out2.reshape(_L, _B // _RW, _E, _RW)
    outputs = o4.transpose(1, 3, 0, 2).reshape(_B, _L, _E)
    loss = _tc_loss(part)[0, 0]
    return (loss, outputs)


# speedup vs baseline: 2.2692x; 1.1621x over previous
import functools

import jax
import jax.numpy as jnp
from jax import lax
from jax.experimental import pallas as pl
from jax.experimental.pallas import tpu as pltpu
from jax.experimental.pallas import tpu_sc as plsc

_B, _L, _E, _V = 16384, 200, 8, 10
_TOT = _B * _L
_RW = 128
_NROW = _TOT // _RW
_NC, _NS = 2, 16
_NW = _NC * _NS
_RPW = _NROW // _NW
_C = 40
_NCH = _RPW // _C

_mesh = plsc.VectorSubcoreMesh(core_axis_name="c", subcore_axis_name="s")


@functools.partial(
    pl.kernel,
    out_type=[
        jax.ShapeDtypeStruct((_NROW * _E, _RW), jnp.float32),
        jax.ShapeDtypeStruct((_NW, 16), jnp.float32),
    ],
    mesh=_mesh,
    compiler_params=pltpu.CompilerParams(
        needs_layout_passes=False, use_tc_tiling_on_sc=False),
    scratch_types=[
        pltpu.VMEM((_C, _RW), jnp.int32),
        pltpu.VMEM((_C, _RW), jnp.int32),
        pltpu.VMEM((_C * _E, _RW), jnp.float32),
        pltpu.VMEM((_C * _E, _RW), jnp.float32),
        pltpu.VMEM((_V * _E * 16,), jnp.float32),
        pltpu.VMEM((_V * 16,), jnp.float32),
        pltpu.VMEM((16,), jnp.float32),
        pltpu.SemaphoreType.DMA,
        pltpu.SemaphoreType.DMA,
        pltpu.SemaphoreType.DMA,
        pltpu.SemaphoreType.DMA,
    ],
)
def _sc_gather(ids_hbm, wbank_hbm, rsbank_hbm, out_hbm, part_hbm,
               idx0, idx1, rows0, rows1, wbank, rsbank, acc_v,
               sem_i0, sem_i1, sem_o0, sem_o1):
    cid = lax.axis_index("c")
    sid = lax.axis_index("s")
    wid = sid * _NC + cid
    base = wid * _RPW

    idx = (idx0, idx1)
    rows = (rows0, rows1)
    sem_i = (sem_i0, sem_i1)
    sem_o = (sem_o0, sem_o1)

    lanes = lax.iota(jnp.int32, 16)
    pltpu.sync_copy(wbank_hbm, wbank)
    pltpu.sync_copy(rsbank_hbm, rsbank)

    def ids_start(i, b):
        pltpu.async_copy(ids_hbm.at[pl.ds(base + i * _C, _C)], idx[b], sem_i[b])

    def ids_wait(b):
        pltpu.make_async_copy(
            ids_hbm.at[pl.ds(0, _C)], idx[b], sem_i[b]).wait()

    def out_start(i, b):
        pltpu.async_copy(rows[b],
                         out_hbm.at[pl.ds((base + i * _C) * _E, _C * _E)],
                         sem_o[b])

    def out_wait(b):
        pltpu.make_async_copy(
            rows[b], out_hbm.at[pl.ds(0, _C * _E)], sem_o[b]).wait()

    def chunk_mid(b, acc):
        ids_wait(b)

        def jbody(j, a):
            for k in range(_RW // 16):
                iv = idx[b][j, pl.ds(k * 16, 16)]
                a = a + plsc.load_gather(rsbank, [iv * 16 + lanes])
                iv128 = iv * (_E * 16)
                for e in range(_E):
                    v = plsc.load_gather(wbank, [iv128 + (lanes + e * 16)])
                    rows[b][j * _E + e, pl.ds(k * 16, 16)] = v
            return a

        return plsc.parallel_loop(0, _C, 1, carry=acc)(jbody)

    ids_start(0, 0)
    ids_start(1, 1)
    acc = jnp.zeros((16,), jnp.float32)
    for b in (0, 1):
        acc = chunk_mid(b, acc)
        ids_start(b + 2, b)
        out_start(b, b)

    def macro(m, acc):
        for b in (0, 1):
            i = 2 * m + b
            out_wait(b)
            acc = chunk_mid(b, acc)
            nxt = jnp.where(i + 2 < _NCH, i + 2, 0)
            ids_start(nxt, b)
            out_start(i, b)
        return acc

    acc = lax.fori_loop(1, _NCH // 2, macro, acc)

    for b in (0, 1):
        ids_wait(b)
        out_wait(b)

    acc_v[...] = acc
    pltpu.sync_copy(acc_v, part_hbm.at[wid])


def _loss_body(p_ref, o_ref):
    o_ref[0, 0] = jnp.sum(p_ref[...]) * (1.0 / (_TOT * _E))


_tc_loss = pl.pallas_call(
    _loss_body,
    out_shape=jax.ShapeDtypeStruct((1, 1), jnp.float32),
    out_specs=pl.BlockSpec(memory_space=pltpu.SMEM),
)


def kernel(input_ids, W):
    ids = input_ids.astype(jnp.int32).T.reshape(_NROW, _RW)
    w = W.astype(jnp.float32)
    wbank = jnp.repeat(w.reshape(-1), 16)
    rsbank = jnp.repeat(w.sum(axis=1), 16)
    out2, part = _sc_gather(ids, wbank, rsbank)
    o4 = out2.reshape(_L, _B // _RW, _E, _RW)
    outputs = o4.transpose(1, 3, 0, 2).reshape(_B, _L, _E)
    loss = _tc_loss(part)[0, 0]
    return (loss, outputs)


# speedup vs baseline: 2.3879x; 1.0523x over previous
import functools

import jax
import jax.numpy as jnp
from jax import lax
from jax.experimental import pallas as pl
from jax.experimental.pallas import tpu as pltpu
from jax.experimental.pallas import tpu_sc as plsc

_B, _L, _E, _V = 16384, 200, 8, 10
_TOT = _B * _L
_RW = 128
_NROW = _TOT // _RW
_NC, _NS = 2, 16
_NW = _NC * _NS
_RPW = _NROW // _NW
_C = 50
_NCH = _RPW // _C

_mesh = plsc.VectorSubcoreMesh(core_axis_name="c", subcore_axis_name="s")


@functools.partial(
    pl.kernel,
    out_type=[
        jax.ShapeDtypeStruct((_NROW * _E, _RW), jnp.float32),
        jax.ShapeDtypeStruct((_NW, 16), jnp.float32),
    ],
    mesh=_mesh,
    compiler_params=pltpu.CompilerParams(
        needs_layout_passes=False, use_tc_tiling_on_sc=False),
    scratch_types=[
        pltpu.VMEM((_C, _RW), jnp.int32),
        pltpu.VMEM((_C, _RW), jnp.int32),
        pltpu.VMEM((_C * _E, _RW), jnp.float32),
        pltpu.VMEM((_C * _E, _RW), jnp.float32),
        pltpu.VMEM((_V * _E * 16,), jnp.float32),
        pltpu.VMEM((_V * 16,), jnp.float32),
        pltpu.VMEM((16,), jnp.float32),
        pltpu.SemaphoreType.DMA,
        pltpu.SemaphoreType.DMA,
        pltpu.SemaphoreType.DMA,
        pltpu.SemaphoreType.DMA,
    ],
)
def _sc_gather(ids_hbm, wbank_hbm, rsbank_hbm, out_hbm, part_hbm,
               idx0, idx1, rows0, rows1, wbank, rsbank, acc_v,
               sem_i0, sem_i1, sem_o0, sem_o1):
    cid = lax.axis_index("c")
    sid = lax.axis_index("s")
    wid = sid * _NC + cid
    base = wid * _RPW

    idx = (idx0, idx1)
    rows = (rows0, rows1)
    sem_i = (sem_i0, sem_i1)
    sem_o = (sem_o0, sem_o1)

    lanes = lax.iota(jnp.int32, 16)
    pltpu.sync_copy(wbank_hbm, wbank)
    pltpu.sync_copy(rsbank_hbm, rsbank)

    def ids_start(i, b):
        pltpu.async_copy(ids_hbm.at[pl.ds(base + i * _C, _C)], idx[b], sem_i[b])

    def ids_wait(b):
        pltpu.make_async_copy(
            ids_hbm.at[pl.ds(0, _C)], idx[b], sem_i[b]).wait()

    def out_start(i, b):
        pltpu.async_copy(rows[b],
                         out_hbm.at[pl.ds((base + i * _C) * _E, _C * _E)],
                         sem_o[b])

    def out_wait(b):
        pltpu.make_async_copy(
            rows[b], out_hbm.at[pl.ds(0, _C * _E)], sem_o[b]).wait()

    def chunk_mid(b, acc):
        ids_wait(b)

        def jbody(j, a):
            for k in range(_RW // 16):
                iv = idx[b][j, pl.ds(k * 16, 16)]
                a = a + plsc.load_gather(rsbank, [iv * 16 + lanes])
                iv128 = iv * (_E * 16)
                for e in range(_E):
                    v = plsc.load_gather(wbank, [iv128 + (lanes + e * 16)])
                    rows[b][j * _E + e, pl.ds(k * 16, 16)] = v
            return a

        return plsc.parallel_loop(0, _C, 1, carry=acc)(jbody)

    ids_start(0, 0)
    ids_start(1, 1)
    acc = jnp.zeros((16,), jnp.float32)
    for b in (0, 1):
        acc = chunk_mid(b, acc)
        ids_start(b + 2, b)
        out_start(b, b)

    def macro(m, acc):
        for b in (0, 1):
            i = 2 * m + b
            out_wait(b)
            acc = chunk_mid(b, acc)
            nxt = jnp.where(i + 2 < _NCH, i + 2, 0)
            ids_start(nxt, b)
            out_start(i, b)
        return acc

    acc = lax.fori_loop(1, _NCH // 2, macro, acc)

    for b in (0, 1):
        ids_wait(b)
        out_wait(b)

    acc_v[...] = acc
    pltpu.sync_copy(acc_v, part_hbm.at[wid])


def _loss_body(p_ref, o_ref):
    o_ref[0, 0] = jnp.sum(p_ref[...]) * (1.0 / (_TOT * _E))


_tc_loss = pl.pallas_call(
    _loss_body,
    out_shape=jax.ShapeDtypeStruct((1, 1), jnp.float32),
    out_specs=pl.BlockSpec(memory_space=pltpu.SMEM),
)


def kernel(input_ids, W):
    ids = input_ids.astype(jnp.int32).T.reshape(_NROW, _RW)
    w = W.astype(jnp.float32)
    wbank = jnp.repeat(w.reshape(-1), 16)
    rsbank = jnp.repeat(w.sum(axis=1), 16)
    out2, part = _sc_gather(ids, wbank, rsbank)
    o4 = out2.reshape(_L, _B // _RW, _E, _RW)
    outputs = o4.transpose(1, 3, 0, 2).reshape(_B, _L, _E)
    loss = _tc_loss(part)[0, 0]
    return (loss, outputs)


# speedup vs baseline: 2.4897x; 1.0426x over previous
import functools

import jax
import jax.numpy as jnp
from jax import lax
from jax.experimental import pallas as pl
from jax.experimental.pallas import tpu as pltpu
from jax.experimental.pallas import tpu_sc as plsc

_B, _L, _E, _V = 16384, 200, 8, 10
_TOT = _B * _L
_RW = 128
_NROW = _TOT // _RW
_NC, _NS = 2, 16
_NW = _NC * _NS
_RPW = _NROW // _NW
_C = 50
_NCH = _RPW // _C

_mesh = plsc.VectorSubcoreMesh(core_axis_name="c", subcore_axis_name="s")


@functools.partial(
    pl.kernel,
    out_type=[
        jax.ShapeDtypeStruct((_NROW * _E, _RW), jnp.float32),
        jax.ShapeDtypeStruct((_NW, 16), jnp.float32),
    ],
    mesh=_mesh,
    compiler_params=pltpu.CompilerParams(
        needs_layout_passes=False, use_tc_tiling_on_sc=False),
    scratch_types=[
        pltpu.VMEM((_C, _RW), jnp.int32),
        pltpu.VMEM((_C, _RW), jnp.int32),
        pltpu.VMEM((_C * _E, _RW), jnp.float32),
        pltpu.VMEM((_C * _E, _RW), jnp.float32),
        pltpu.VMEM((_V * _E * 16,), jnp.float32),
        pltpu.VMEM((_V * 16,), jnp.float32),
        pltpu.VMEM((16,), jnp.float32),
        pltpu.SemaphoreType.DMA,
        pltpu.SemaphoreType.DMA,
        pltpu.SemaphoreType.DMA,
        pltpu.SemaphoreType.DMA,
    ],
)
def _sc_gather(ids_hbm, wbank_hbm, rsbank_hbm, out_hbm, part_hbm,
               idx0, idx1, rows0, rows1, wbank, rsbank, acc_v,
               sem_i0, sem_i1, sem_o0, sem_o1):
    cid = lax.axis_index("c")
    sid = lax.axis_index("s")
    wid = sid * _NC + cid
    base = wid * _RPW

    idx = (idx0, idx1)
    rows = (rows0, rows1)
    sem_i = (sem_i0, sem_i1)
    sem_o = (sem_o0, sem_o1)

    lanes = lax.iota(jnp.int32, 16)
    pltpu.sync_copy(wbank_hbm, wbank)
    pltpu.sync_copy(rsbank_hbm, rsbank)

    def ids_start(i, b):
        pltpu.async_copy(ids_hbm.at[pl.ds(base + i * _C, _C)], idx[b], sem_i[b])

    def ids_wait(b):
        pltpu.make_async_copy(
            ids_hbm.at[pl.ds(0, _C)], idx[b], sem_i[b]).wait()

    def out_start(i, b):
        pltpu.async_copy(rows[b],
                         out_hbm.at[pl.ds((base + i * _C) * _E, _C * _E)],
                         sem_o[b])

    def out_wait(b):
        pltpu.make_async_copy(
            rows[b], out_hbm.at[pl.ds(0, _C * _E)], sem_o[b]).wait()

    def chunk_mid(b, acc):
        ids_wait(b)

        def jbody(j, a):
            for k in range(_RW // 16):
                iv = idx[b][j, pl.ds(k * 16, 16)]
                a = a + plsc.load_gather(rsbank, [iv * 16 + lanes])
                iv128 = iv * (_E * 16)
                for e in range(_E):
                    v = plsc.load_gather(wbank, [iv128 + (lanes + e * 16)])
                    rows[b][j * _E + e, pl.ds(k * 16, 16)] = v
            return a

        acc = plsc.parallel_loop(0, _C, 1, carry=acc)(jbody)
        plsc.subcore_barrier()
        return acc

    ids_start(0, 0)
    ids_start(1, 1)
    acc = jnp.zeros((16,), jnp.float32)
    for b in (0, 1):
        acc = chunk_mid(b, acc)
        ids_start(b + 2, b)
        out_start(b, b)

    def macro(m, acc):
        for b in (0, 1):
            i = 2 * m + b
            out_wait(b)
            acc = chunk_mid(b, acc)
            nxt = jnp.where(i + 2 < _NCH, i + 2, 0)
            ids_start(nxt, b)
            out_start(i, b)
        return acc

    acc = lax.fori_loop(1, _NCH // 2, macro, acc)

    for b in (0, 1):
        ids_wait(b)
        out_wait(b)

    acc_v[...] = acc
    pltpu.sync_copy(acc_v, part_hbm.at[wid])


def _loss_body(p_ref, o_ref):
    o_ref[0, 0] = jnp.sum(p_ref[...]) * (1.0 / (_TOT * _E))


_tc_loss = pl.pallas_call(
    _loss_body,
    out_shape=jax.ShapeDtypeStruct((1, 1), jnp.float32),
    out_specs=pl.BlockSpec(memory_space=pltpu.SMEM),
)


def kernel(input_ids, W):
    ids = input_ids.astype(jnp.int32).T.reshape(_NROW, _RW)
    w = W.astype(jnp.float32)
    wbank = jnp.repeat(w.reshape(-1), 16)
    rsbank = jnp.repeat(w.sum(axis=1), 16)
    out2, part = _sc_gather(ids, wbank, rsbank)
    o4 = out2.reshape(_L, _B // _RW, _E, _RW)
    outputs = o4.transpose(1, 3, 0, 2).reshape(_B, _L, _E)
    loss = _tc_loss(part)[0, 0]
    return (loss, outputs)
